# Initial kernel scaffold; baseline (speedup 1.0000x reference)
import jax
import jax.numpy as jnp
from jax.experimental import pallas as pl


def kernel(coords, embedding):
    raise NotImplementedError("write your pallas kernel here")


# speedup vs baseline: 9.0952x; 9.0952x over previous
import functools

import jax
import jax.numpy as jnp
from jax import lax
from jax.experimental import pallas as pl
from jax.experimental.pallas import tpu as pltpu
from jax.experimental.pallas import tpu_sc as plsc

GRID = 64
DIM = 64
LANES = 16
NC = 2
NS = 16
NW = NC * NS
DCH = DIM // LANES
NPG = NW // DCH
CHUNK = 1024


def _make_interp(n_points: int):
    ppg = n_points // NPG
    iters = ppg // CHUNK
    mesh = plsc.VectorSubcoreMesh(core_axis_name="c", subcore_axis_name="s")

    @functools.partial(
        pl.kernel,
        mesh=mesh,
        out_type=jax.ShapeDtypeStruct((n_points, DIM), jnp.float32),
        scratch_types=[
            pltpu.VMEM((GRID * GRID * LANES,), jnp.float32),
            pltpu.VMEM((2 * CHUNK,), jnp.float32),
            pltpu.VMEM((CHUNK, LANES), jnp.float32),
        ],
        compiler_params=pltpu.CompilerParams(
            use_tc_tiling_on_sc=False, needs_layout_passes=False
        ),
    )
    def interp(coords_hbm, table_hbm, out_hbm, tab_v, crd_v, out_v):
        wid = lax.axis_index("s") * NC + lax.axis_index("c")
        g = wid % DCH
        pg = wid // DCH

        pltpu.sync_copy(table_hbm.at[g], tab_v)

        lanes = lax.iota(jnp.int32, LANES)
        ob_lane = lanes

        def chunk_body(it, carry):
            p0 = pg * ppg + it * CHUNK
            pltpu.sync_copy(coords_hbm.at[pl.ds(p0 * 2, 2 * CHUNK)], crd_v)

            def sub_body(s, c2):
                idx2 = lanes * 2 + s * (2 * LANES)
                xs = plsc.load_gather(crd_v, [idx2])
                ys = plsc.load_gather(crd_v, [idx2 + 1])
                cx = xs * jnp.float32(GRID - 1)
                cy = ys * jnp.float32(GRID - 1)
                xi = jnp.minimum(jnp.maximum(cx.astype(jnp.int32), 0), GRID - 2)
                yi = jnp.minimum(jnp.maximum(cy.astype(jnp.int32), 0), GRID - 2)
                fx = cx - xi.astype(jnp.float32)
                fy = cy - yi.astype(jnp.float32)
                gx = 1.0 - fx
                gy = 1.0 - fy
                w00 = gx * gy
                w01 = gx * fy
                w10 = fx * gy
                w11 = fx * fy
                e00 = (xi * GRID + yi) * LANES
                row = ob_lane + s * LANES
                for j in range(LANES):
                    v00 = plsc.load_gather(tab_v, [e00 + j])
                    v01 = plsc.load_gather(tab_v, [e00 + (LANES + j)])
                    v10 = plsc.load_gather(tab_v, [e00 + (GRID * LANES + j)])
                    v11 = plsc.load_gather(tab_v, [e00 + (GRID * LANES + LANES + j)])
                    r = v00 * w00 + v01 * w01 + v10 * w10 + v11 * w11
                    col = jnp.full((LANES,), j, dtype=jnp.int32)
                    plsc.store_scatter(out_v, [row, col], r)
                return c2

            lax.fori_loop(0, CHUNK // LANES, sub_body, 0, unroll=False)
            pltpu.sync_copy(
                out_v, out_hbm.at[pl.ds(p0, CHUNK), pl.ds(g * LANES, LANES)]
            )
            return carry

        lax.fori_loop(0, iters, chunk_body, 0, unroll=False)

    return interp


def kernel(coords, embedding):
    b, l, _ = coords.shape
    n = b * l
    assert embedding.shape == (GRID, GRID, DIM)
    assert n % (NPG * CHUNK) == 0
    cflat = coords.reshape(n * 2)
    table = (
        embedding.reshape(GRID * GRID, DCH, LANES)
        .transpose(1, 0, 2)
        .reshape(DCH, GRID * GRID * LANES)
    )
    out = _make_interp(n)(cflat, table)
    return out.reshape(b, l, DIM)


# speedup vs baseline: 12.2411x; 1.3459x over previous
import functools

import jax
import jax.numpy as jnp
from jax import lax
from jax.experimental import pallas as pl
from jax.experimental.pallas import tpu as pltpu
from jax.experimental.pallas import tpu_sc as plsc

GRID = 64
DIM = 64
LANES = 16
NC = 2
NS = 16
NW = NC * NS
DCH = DIM // LANES
NPG = NW // DCH
CHUNK = 1024


def _make_interp(n_points: int):
    ppg = n_points // NPG
    iters = ppg // CHUNK
    mesh = plsc.VectorSubcoreMesh(core_axis_name="c", subcore_axis_name="s")

    @functools.partial(
        pl.kernel,
        mesh=mesh,
        out_type=jax.ShapeDtypeStruct((n_points, DIM), jnp.float32),
        scratch_types=[
            pltpu.VMEM((GRID * GRID * LANES,), jnp.float32),
            pltpu.VMEM((2 * CHUNK,), jnp.float32),
            pltpu.VMEM((CHUNK, LANES), jnp.float32),
        ],
        compiler_params=pltpu.CompilerParams(
            use_tc_tiling_on_sc=False, needs_layout_passes=False
        ),
    )
    def interp(coords_hbm, table_hbm, out_hbm, tab_v, crd_v, out_v):
        wid = lax.axis_index("s") * NC + lax.axis_index("c")
        g = wid % DCH
        pg = wid // DCH

        pltpu.sync_copy(table_hbm.at[g], tab_v)

        lanes = lax.iota(jnp.int32, LANES)
        ob_lane = lanes

        def chunk_body(it, carry):
            p0 = pg * ppg + it * CHUNK
            pltpu.sync_copy(coords_hbm.at[pl.ds(p0 * 2, 2 * CHUNK)], crd_v)

            @plsc.parallel_loop(0, CHUNK // LANES, 1)
            def sub_body(s):
                idx2 = lanes * 2 + s * (2 * LANES)
                xs = plsc.load_gather(crd_v, [idx2])
                ys = plsc.load_gather(crd_v, [idx2 + 1])
                cx = xs * jnp.float32(GRID - 1)
                cy = ys * jnp.float32(GRID - 1)
                xi = jnp.minimum(jnp.maximum(cx.astype(jnp.int32), 0), GRID - 2)
                yi = jnp.minimum(jnp.maximum(cy.astype(jnp.int32), 0), GRID - 2)
                fx = cx - xi.astype(jnp.float32)
                fy = cy - yi.astype(jnp.float32)
                gx = 1.0 - fx
                gy = 1.0 - fy
                w00 = gx * gy
                w01 = gx * fy
                w10 = fx * gy
                w11 = fx * fy
                e00 = (xi * GRID + yi) * LANES
                row = ob_lane + s * LANES
                for j in range(LANES):
                    v00 = plsc.load_gather(tab_v, [e00 + j])
                    v01 = plsc.load_gather(tab_v, [e00 + (LANES + j)])
                    v10 = plsc.load_gather(tab_v, [e00 + (GRID * LANES + j)])
                    v11 = plsc.load_gather(tab_v, [e00 + (GRID * LANES + LANES + j)])
                    r = v00 * w00 + v01 * w01 + v10 * w10 + v11 * w11
                    col = jnp.full((LANES,), j, dtype=jnp.int32)
                    plsc.store_scatter(out_v, [row, col], r)

            pltpu.sync_copy(
                out_v, out_hbm.at[pl.ds(p0, CHUNK), pl.ds(g * LANES, LANES)]
            )
            return carry

        lax.fori_loop(0, iters, chunk_body, 0, unroll=False)

    return interp


def kernel(coords, embedding):
    b, l, _ = coords.shape
    n = b * l
    assert embedding.shape == (GRID, GRID, DIM)
    assert n % (NPG * CHUNK) == 0
    cflat = coords.reshape(n * 2)
    table = (
        embedding.reshape(GRID * GRID, DCH, LANES)
        .transpose(1, 0, 2)
        .reshape(DCH, GRID * GRID * LANES)
    )
    out = _make_interp(n)(cflat, table)
    return out.reshape(b, l, DIM)


# speedup vs baseline: 13.0286x; 1.0643x over previous
import functools

import jax
import jax.numpy as jnp
from jax import lax
from jax.experimental import pallas as pl
from jax.experimental.pallas import tpu as pltpu
from jax.experimental.pallas import tpu_sc as plsc

GRID = 64
DIM = 64
LANES = 16
NC = 2
NS = 16
NW = NC * NS
DCH = DIM // LANES
NPG = NW // DCH
CHUNK = 1024
NBUF = 2


def _make_interp(n_points: int):
    ppg = n_points // NPG
    iters = ppg // CHUNK
    assert iters % NBUF == 0
    mesh = plsc.VectorSubcoreMesh(core_axis_name="c", subcore_axis_name="s")

    @functools.partial(
        pl.kernel,
        mesh=mesh,
        out_type=jax.ShapeDtypeStruct((n_points, DIM), jnp.float32),
        scratch_types=[
            pltpu.VMEM((GRID * GRID * LANES,), jnp.float32),
            [pltpu.VMEM((2 * CHUNK,), jnp.float32) for _ in range(NBUF)],
            [pltpu.VMEM((CHUNK, LANES), jnp.float32) for _ in range(NBUF)],
            [pltpu.SemaphoreType.DMA for _ in range(NBUF)],
            [pltpu.SemaphoreType.DMA for _ in range(NBUF)],
        ],
        compiler_params=pltpu.CompilerParams(
            use_tc_tiling_on_sc=False, needs_layout_passes=False
        ),
    )
    def interp(coords_hbm, table_hbm, out_hbm, tab_v, crd_v, out_v, sin, sout):
        wid = lax.axis_index("s") * NC + lax.axis_index("c")
        g = wid % DCH
        pg = wid // DCH
        p_base = pg * ppg

        pltpu.sync_copy(table_hbm.at[g], tab_v)

        lanes = lax.iota(jnp.int32, LANES)

        def in_copy(it, b):
            p0 = p_base + it * CHUNK
            return pltpu.make_async_copy(
                coords_hbm.at[pl.ds(p0 * 2, 2 * CHUNK)], crd_v[b], sin[b]
            )

        def out_copy(it, b):
            p0 = p_base + it * CHUNK
            return pltpu.make_async_copy(
                out_v[b],
                out_hbm.at[pl.ds(p0, CHUNK), pl.ds(g * LANES, LANES)],
                sout[b],
            )

        def compute(it, b):
            @plsc.parallel_loop(0, CHUNK // LANES, 1)
            def sub_body(s):
                idx2 = lanes * 2 + s * (2 * LANES)
                xs = plsc.load_gather(crd_v[b], [idx2])
                ys = plsc.load_gather(crd_v[b], [idx2 + 1])
                cx = xs * jnp.float32(GRID - 1)
                cy = ys * jnp.float32(GRID - 1)
                xi = jnp.minimum(jnp.maximum(cx.astype(jnp.int32), 0), GRID - 2)
                yi = jnp.minimum(jnp.maximum(cy.astype(jnp.int32), 0), GRID - 2)
                fx = cx - xi.astype(jnp.float32)
                fy = cy - yi.astype(jnp.float32)
                gx = 1.0 - fx
                gy = 1.0 - fy
                w00 = gx * gy
                w01 = gx * fy
                w10 = fx * gy
                w11 = fx * fy
                e00 = (xi * GRID + yi) * LANES
                row = lanes + s * LANES
                for j in range(LANES):
                    v00 = plsc.load_gather(tab_v, [e00 + j])
                    v01 = plsc.load_gather(tab_v, [e00 + (LANES + j)])
                    v10 = plsc.load_gather(tab_v, [e00 + (GRID * LANES + j)])
                    v11 = plsc.load_gather(tab_v, [e00 + (GRID * LANES + LANES + j)])
                    r = v00 * w00 + v01 * w01 + v10 * w10 + v11 * w11
                    col = jnp.full((LANES,), j, dtype=jnp.int32)
                    plsc.store_scatter(out_v[b], [row, col], r)

        for b in range(NBUF):
            in_copy(b, b).start()

        def stage(it2, carry):
            for b in range(NBUF):
                it = it2 * NBUF + b
                in_copy(it, b).wait()

                @pl.when(it2 >= 1)
                def _drain():
                    out_copy(it - NBUF, b).wait()

                compute(it, b)
                out_copy(it, b).start()

                @pl.when(it + NBUF < iters)
                def _prefetch():
                    in_copy(it + NBUF, b).start()

            return carry

        lax.fori_loop(0, iters // NBUF, stage, 0, unroll=False)
        for b in range(NBUF):
            out_copy(iters - NBUF + b, b).wait()

    return interp


def kernel(coords, embedding):
    b, l, _ = coords.shape
    n = b * l
    assert embedding.shape == (GRID, GRID, DIM)
    assert n % (NPG * CHUNK) == 0
    cflat = coords.reshape(n * 2)
    table = (
        embedding.reshape(GRID * GRID, DCH, LANES)
        .transpose(1, 0, 2)
        .reshape(DCH, GRID * GRID * LANES)
    )
    out = _make_interp(n)(cflat, table)
    return out.reshape(b, l, DIM)


# speedup vs baseline: 16.6146x; 1.2752x over previous
import functools

import jax
import jax.numpy as jnp
from jax import lax
from jax.experimental import pallas as pl
from jax.experimental.pallas import tpu as pltpu
from jax.experimental.pallas import tpu_sc as plsc

GRID = 64
DIM = 64
LANES = 16
NC = 2
NS = 16
NW = NC * NS
DCH = DIM // LANES
NPG = NW // DCH
CHUNK = 1024
NBUF = 2


def _make_interp(n_points: int):
    ppg = n_points // NPG
    iters = ppg // CHUNK
    assert iters % NBUF == 0
    mesh = plsc.VectorSubcoreMesh(core_axis_name="c", subcore_axis_name="s")

    @functools.partial(
        pl.kernel,
        mesh=mesh,
        out_type=jax.ShapeDtypeStruct((n_points, DIM), jnp.float32),
        scratch_types=[
            pltpu.VMEM((GRID * GRID * LANES,), jnp.float32),
            [pltpu.VMEM((2 * CHUNK,), jnp.float32) for _ in range(NBUF)],
            [pltpu.VMEM((CHUNK, LANES), jnp.float32) for _ in range(NBUF)],
            [pltpu.SemaphoreType.DMA for _ in range(NBUF)],
            [pltpu.SemaphoreType.DMA for _ in range(NBUF)],
        ],
        compiler_params=pltpu.CompilerParams(
            use_tc_tiling_on_sc=False, needs_layout_passes=False
        ),
    )
    def interp(coords_hbm, table_hbm, out_hbm, tab_v, crd_v, out_v, sin, sout):
        wid = lax.axis_index("s") * NC + lax.axis_index("c")
        g = wid % DCH
        pg = wid // DCH
        p_base = pg * ppg

        pltpu.sync_copy(table_hbm.at[g], tab_v)

        lanes = lax.iota(jnp.int32, LANES)

        def in_copy(it, b):
            p0 = p_base + it * CHUNK
            return pltpu.make_async_copy(
                coords_hbm.at[pl.ds(p0 * 2, 2 * CHUNK)], crd_v[b], sin[b]
            )

        def out_copy(it, b):
            p0 = p_base + it * CHUNK
            return pltpu.make_async_copy(
                out_v[b],
                out_hbm.at[pl.ds(p0, CHUNK), pl.ds(g * LANES, LANES)],
                sout[b],
            )

        def compute(it, b):
            @plsc.parallel_loop(0, CHUNK // LANES, 1, unroll=8)
            def sub_body(s):
                idx2 = lanes * 2 + s * (2 * LANES)
                xs = plsc.load_gather(crd_v[b], [idx2])
                ys = plsc.load_gather(crd_v[b], [idx2 + 1])
                cx = xs * jnp.float32(GRID - 1)
                cy = ys * jnp.float32(GRID - 1)
                xi = jnp.minimum(jnp.maximum(cx.astype(jnp.int32), 0), GRID - 2)
                yi = jnp.minimum(jnp.maximum(cy.astype(jnp.int32), 0), GRID - 2)
                fx = cx - xi.astype(jnp.float32)
                fy = cy - yi.astype(jnp.float32)
                gx = 1.0 - fx
                gy = 1.0 - fy
                w00 = gx * gy
                w01 = gx * fy
                w10 = fx * gy
                w11 = fx * fy
                e00 = (xi * GRID + yi) * LANES
                row = lanes + s * LANES
                for j in range(LANES):
                    rot = (lanes + j) & (LANES - 1)
                    v00 = plsc.load_gather(tab_v, [e00 + rot])
                    v01 = plsc.load_gather(tab_v, [e00 + (rot + LANES)])
                    v10 = plsc.load_gather(tab_v, [e00 + (rot + GRID * LANES)])
                    v11 = plsc.load_gather(
                        tab_v, [e00 + (rot + (GRID * LANES + LANES))]
                    )
                    r = v00 * w00 + v01 * w01 + v10 * w10 + v11 * w11
                    plsc.store_scatter(out_v[b], [row, rot], r)

        for b in range(NBUF):
            in_copy(b, b).start()

        def stage(it2, carry):
            for b in range(NBUF):
                it = it2 * NBUF + b
                in_copy(it, b).wait()

                @pl.when(it2 >= 1)
                def _drain():
                    out_copy(it - NBUF, b).wait()

                compute(it, b)
                out_copy(it, b).start()

                @pl.when(it + NBUF < iters)
                def _prefetch():
                    in_copy(it + NBUF, b).start()

            return carry

        lax.fori_loop(0, iters // NBUF, stage, 0, unroll=False)
        for b in range(NBUF):
            out_copy(iters - NBUF + b, b).wait()

    return interp


def kernel(coords, embedding):
    b, l, _ = coords.shape
    n = b * l
    assert embedding.shape == (GRID, GRID, DIM)
    assert n % (NPG * CHUNK) == 0
    cflat = coords.reshape(n * 2)
    table = (
        embedding.reshape(GRID * GRID, DCH, LANES)
        .transpose(1, 0, 2)
        .reshape(DCH, GRID * GRID * LANES)
    )
    out = _make_interp(n)(cflat, table)
    return out.reshape(b, l, DIM)
